# baseline (speedup 1.0000x reference)
import functools

import jax
import jax.numpy as jnp
from jax import lax
from jax.experimental import pallas as pl
from jax.experimental.pallas import tpu as pltpu
from jax.experimental.pallas import tpu_sc as plsc

_ROWS = 1024
_COLS = 32768
_LANES = 16
_NC = 2
_NS = 16
_NW = _NC * _NS
_RPW = _ROWS // _NW
_CHUNKS = _COLS // _LANES


def _row_argmax(row_buf, parity, lanes):
    n_acc = 8
    group = 16
    m0 = [jnp.full((_LANES,), -jnp.inf, jnp.float32) for _ in range(n_acc)]
    b0 = [jnp.zeros((_LANES,), jnp.int32) for _ in range(n_acc)]

    @plsc.parallel_loop(0, _COLS, step=group * _LANES, carry=(m0, b0))
    def carry_out(off, carry):
        ms, bs = carry
        ms, bs = list(ms), list(bs)
        j = off // _LANES
        for k in range(group):
            a = k % n_acc
            v = row_buf[parity, pl.ds(off + k * _LANES, _LANES)]
            take = v > ms[a]
            ms[a] = jnp.where(take, v, ms[a])
            bs[a] = jnp.where(take, jnp.full((_LANES,), j + k, jnp.int32),
                              bs[a])
        return ms, bs

    ms, bs = carry_out

    def merge(m1, b1, m2, b2):
        take = (m2 > m1) | ((m2 == m1) & (b2 < b1))
        return jnp.where(take, m2, m1), jnp.where(take, b2, b1)

    while len(ms) > 1:
        ms = [merge(ms[2 * i], bs[2 * i], ms[2 * i + 1], bs[2 * i + 1])
              for i in range(len(ms) // 2)]
        ms, bs = [p[0] for p in ms], [p[1] for p in ms]
    m, bj = ms[0], bs[0]
    col = bj * _LANES + lanes
    gm = jnp.max(m)
    cand = jnp.where(m == gm, col, jnp.int32(2**30))
    return jnp.min(cand).astype(jnp.float32)


def _argmax_rows_sc(x_flat):
    mesh = plsc.VectorSubcoreMesh(
        core_axis_name="c", subcore_axis_name="s",
        num_cores=_NC, num_subcores=_NS)

    @functools.partial(
        pl.kernel,
        out_type=jax.ShapeDtypeStruct((_ROWS,), jnp.float32),
        mesh=mesh,
        scratch_types=[
            pltpu.VMEM((2, _COLS), jnp.float32),
            pltpu.VMEM((_RPW,), jnp.float32),
            pltpu.SemaphoreType.DMA,
            pltpu.SemaphoreType.DMA,
        ],
        compiler_params=pltpu.CompilerParams(needs_layout_passes=False),
    )
    def k(x_hbm, out_hbm, row_buf, out_buf, sem0, sem1):
        wid = lax.axis_index("s") * _NC + lax.axis_index("c")
        base = wid * _RPW
        sems = (sem0, sem1)

        lanes = lax.iota(jnp.int32, _LANES)
        pending = pltpu.async_copy(x_hbm.at[base], row_buf.at[0], sems[0])
        res = jnp.zeros((_LANES,), jnp.float32)
        for r in range(_RPW):
            nxt = None
            if r + 1 < _RPW:
                nxt = pltpu.async_copy(
                    x_hbm.at[base + (r + 1)],
                    row_buf.at[(r + 1) % 2], sems[(r + 1) % 2])
            pending.wait()
            val = _row_argmax(row_buf, r % 2, lanes)
            res = jnp.where(lanes == (r % _LANES), val, res)
            if (r + 1) % _LANES == 0:
                out_buf[pl.ds((r // _LANES) * _LANES, _LANES)] = res
                res = jnp.zeros((_LANES,), jnp.float32)
            pending = nxt
        pltpu.sync_copy(out_buf, out_hbm.at[pl.ds(base, _RPW)])

    return k(x_flat)


def kernel(output):
    flat = output.reshape(_ROWS, _COLS)
    idx = _argmax_rows_sc(flat)
    return idx.reshape(64, 16, 1)


# speedup vs baseline: 1.2979x; 1.2979x over previous
import functools

import jax
import jax.numpy as jnp
from jax import lax
from jax.experimental import pallas as pl
from jax.experimental.pallas import tpu as pltpu
from jax.experimental.pallas import tpu_sc as plsc

_ROWS = 1024
_COLS = 32768
_LANES = 16
_NC = 2
_NS = 16
_NW = _NC * _NS
_RPW = _ROWS // _NW
_CHUNKS = _COLS // _LANES
_NSEG = 32
_SEG_CHUNKS = _CHUNKS // _NSEG
_BIG = 2**30


def _row_argmax(buf, segmax, lanes):
    n_acc = 8
    neg_inf = jnp.full((_LANES,), -jnp.inf, jnp.float32)

    @plsc.parallel_loop(0, _NSEG, carry=None)
    def _(s):
        sbase = s * (_SEG_CHUNKS * _LANES)
        accs = [neg_inf for _ in range(n_acc)]
        for k in range(_SEG_CHUNKS):
            v = buf[pl.ds(sbase + k * _LANES, _LANES)]
            accs[k % n_acc] = jnp.maximum(accs[k % n_acc], v)
        while len(accs) > 1:
            accs = [jnp.maximum(accs[2 * i], accs[2 * i + 1])
                    for i in range(len(accs) // 2)]
        segmax[pl.ds(s * _LANES, _LANES)] = accs[0]

    @plsc.parallel_loop(0, _NSEG, carry=neg_inf)
    def gvec(s, acc):
        return jnp.maximum(acc, segmax[pl.ds(s * _LANES, _LANES)])

    gm = jnp.max(gvec)
    gmv = jnp.full((_LANES,), gm)

    @plsc.parallel_loop(0, _NSEG, carry=jnp.full((_LANES,), _BIG, jnp.int32))
    def run_s(s, acc):
        seg = segmax[pl.ds(s * _LANES, _LANES)]
        sv = jnp.full((_LANES,), s, jnp.int32)
        return jnp.minimum(acc, jnp.where(seg == gmv, sv, _BIG))

    seg_star = jnp.min(run_s)

    cbase = seg_star * (_SEG_CHUNKS * _LANES)
    big_v = jnp.full((_LANES,), _BIG, jnp.int32)

    @plsc.parallel_loop(0, _SEG_CHUNKS, step=2, carry=(big_v, big_v))
    def runs(kk, carry):
        r0, r1 = carry
        v0 = buf[pl.ds(cbase + kk * _LANES, _LANES)]
        v1 = buf[pl.ds(cbase + (kk + 1) * _LANES, _LANES)]
        k0 = jnp.full((_LANES,), kk, jnp.int32)
        r0 = jnp.minimum(r0, jnp.where(v0 == gmv, k0, _BIG))
        r1 = jnp.minimum(r1, jnp.where(v1 == gmv, k0 + 1, _BIG))
        return r0, r1

    runk = jnp.minimum(runs[0], runs[1])
    kcol = jnp.where(runk == _BIG, _BIG, runk * _LANES + lanes)
    col = cbase + jnp.min(kcol)
    return col.astype(jnp.float32)


def _argmax_rows_sc(x_flat):
    mesh = plsc.VectorSubcoreMesh(
        core_axis_name="c", subcore_axis_name="s",
        num_cores=_NC, num_subcores=_NS)

    @functools.partial(
        pl.kernel,
        out_type=jax.ShapeDtypeStruct((_ROWS,), jnp.float32),
        mesh=mesh,
        scratch_types=[
            pltpu.VMEM((_COLS,), jnp.float32),
            pltpu.VMEM((_COLS,), jnp.float32),
            pltpu.VMEM((_NSEG * _LANES,), jnp.float32),
            pltpu.VMEM((_RPW,), jnp.float32),
            pltpu.SemaphoreType.DMA,
            pltpu.SemaphoreType.DMA,
        ],
        compiler_params=pltpu.CompilerParams(needs_layout_passes=False),
    )
    def k(x_hbm, out_hbm, buf0, buf1, segmax, out_buf, sem0, sem1):
        wid = lax.axis_index("s") * _NC + lax.axis_index("c")
        base = wid * _RPW
        lanes = lax.iota(jnp.int32, _LANES)
        bufs = ((buf0, sem0), (buf1, sem1))

        pltpu.async_copy(x_hbm.at[base], buf0, sem0)
        pltpu.async_copy(x_hbm.at[base + 1], buf1, sem1)

        def outer(g, res):
            for b, (buf, sem) in enumerate(bufs):
                r = 2 * g + b
                pltpu.make_async_copy(x_hbm.at[base], buf, sem).wait()
                val = _row_argmax(buf, segmax, lanes)

                @pl.when(r + 2 < _RPW)
                def _():
                    pltpu.async_copy(x_hbm.at[base + r + 2], buf, sem)
                res = jnp.where(lanes == (r % _LANES), val, res)
                flush = (r % _LANES) == (_LANES - 1)

                @pl.when(flush)
                def _():
                    out_buf[pl.ds((r // _LANES) * _LANES, _LANES)] = res

                res = jnp.where(flush, jnp.zeros((_LANES,), jnp.float32),
                                res)
            return res

        lax.fori_loop(0, _RPW // 2, outer, jnp.zeros((_LANES,), jnp.float32))
        pltpu.sync_copy(out_buf, out_hbm.at[pl.ds(base, _RPW)])

    return k(x_flat)


def kernel(output):
    flat = output.reshape(_ROWS, _COLS)
    idx = _argmax_rows_sc(flat)
    return idx.reshape(64, 16, 1)


# speedup vs baseline: 1.3564x; 1.0451x over previous
import functools

import jax
import jax.numpy as jnp
from jax import lax
from jax.experimental import pallas as pl
from jax.experimental.pallas import tpu as pltpu
from jax.experimental.pallas import tpu_sc as plsc

_ROWS = 1024
_COLS = 32768
_LANES = 16
_NC = 2
_NS = 16
_NW = _NC * _NS
_RPW = _ROWS // _NW
_CHUNKS = _COLS // _LANES
_NSEG = 32
_SEG_CHUNKS = _CHUNKS // _NSEG
_BIG = 2**30


def _row_argmax(buf, segmax, lanes):
    n_acc = 8
    neg_inf = jnp.full((_LANES,), -jnp.inf, jnp.float32)

    @plsc.parallel_loop(0, _NSEG, carry=None)
    def _(s):
        sbase = s * (_SEG_CHUNKS * _LANES)
        accs = [neg_inf for _ in range(n_acc)]
        for k in range(_SEG_CHUNKS):
            v = buf[pl.ds(sbase + k * _LANES, _LANES)]
            accs[k % n_acc] = jnp.maximum(accs[k % n_acc], v)
        while len(accs) > 1:
            accs = [jnp.maximum(accs[2 * i], accs[2 * i + 1])
                    for i in range(len(accs) // 2)]
        segmax[pl.ds(s * _LANES, _LANES)] = accs[0]

    @plsc.parallel_loop(0, _NSEG, carry=neg_inf)
    def gvec(s, acc):
        return jnp.maximum(acc, segmax[pl.ds(s * _LANES, _LANES)])

    gm = jnp.max(gvec)
    gmv = jnp.full((_LANES,), gm)

    @plsc.parallel_loop(0, _NSEG, carry=jnp.full((_LANES,), _BIG, jnp.int32))
    def run_s(s, acc):
        seg = segmax[pl.ds(s * _LANES, _LANES)]
        sv = jnp.full((_LANES,), s, jnp.int32)
        return jnp.minimum(acc, jnp.where(seg == gmv, sv, _BIG))

    seg_star = jnp.min(run_s)

    cbase = seg_star * (_SEG_CHUNKS * _LANES)
    big_v = jnp.full((_LANES,), _BIG, jnp.int32)

    @plsc.parallel_loop(0, _SEG_CHUNKS, step=2, carry=(big_v, big_v))
    def runs(kk, carry):
        r0, r1 = carry
        v0 = buf[pl.ds(cbase + kk * _LANES, _LANES)]
        v1 = buf[pl.ds(cbase + (kk + 1) * _LANES, _LANES)]
        k0 = jnp.full((_LANES,), kk, jnp.int32)
        r0 = jnp.minimum(r0, jnp.where(v0 == gmv, k0, _BIG))
        r1 = jnp.minimum(r1, jnp.where(v1 == gmv, k0 + 1, _BIG))
        return r0, r1

    runk = jnp.minimum(runs[0], runs[1])
    kcol = jnp.where(runk == _BIG, _BIG, runk * _LANES + lanes)
    col = cbase + jnp.min(kcol)
    return col.astype(jnp.float32)


def _argmax_rows_sc(x_flat):
    mesh = plsc.VectorSubcoreMesh(
        core_axis_name="c", subcore_axis_name="s",
        num_cores=_NC, num_subcores=_NS)

    @functools.partial(
        pl.kernel,
        out_type=jax.ShapeDtypeStruct((_ROWS,), jnp.float32),
        mesh=mesh,
        scratch_types=[
            pltpu.VMEM((_COLS,), jnp.float32),
            pltpu.VMEM((_COLS,), jnp.float32),
            pltpu.VMEM((_COLS,), jnp.float32),
            pltpu.VMEM((_NSEG * _LANES,), jnp.float32),
            pltpu.VMEM((_RPW,), jnp.float32),
            pltpu.SemaphoreType.DMA,
            pltpu.SemaphoreType.DMA,
            pltpu.SemaphoreType.DMA,
        ],
        compiler_params=pltpu.CompilerParams(needs_layout_passes=False),
    )
    def k(x_hbm, out_hbm, buf0, buf1, buf2, segmax, out_buf, *sems):
        wid = lax.axis_index("s") * _NC + lax.axis_index("c")
        base = wid * _RPW
        lanes = lax.iota(jnp.int32, _LANES)
        bufs = ((buf0, sems[0]), (buf1, sems[1]), (buf2, sems[2]))

        for s, (buf, sem) in enumerate(bufs):
            pltpu.async_copy(x_hbm.at[base + s], buf, sem)

        def one_row(r, res, buf, sem):
            pltpu.make_async_copy(x_hbm.at[base], buf, sem).wait()
            val = _row_argmax(buf, segmax, lanes)

            @pl.when(r + 3 < _RPW)
            def _():
                pltpu.async_copy(x_hbm.at[base + r + 3], buf, sem)

            res = jnp.where(lanes == (r % _LANES), val, res)
            flush = (r % _LANES) == (_LANES - 1)

            @pl.when(flush)
            def _():
                out_buf[pl.ds((r // _LANES) * _LANES, _LANES)] = res

            return jnp.where(flush, jnp.zeros((_LANES,), jnp.float32), res)

        def outer(g, res):
            for b, (buf, sem) in enumerate(bufs):
                res = one_row(3 * g + b, res, buf, sem)
            return res

        res = lax.fori_loop(0, _RPW // 3, outer,
                            jnp.zeros((_LANES,), jnp.float32))
        for r in range((_RPW // 3) * 3, _RPW):
            buf, sem = bufs[r % 3]
            res = one_row(r, res, buf, sem)
        pltpu.sync_copy(out_buf, out_hbm.at[pl.ds(base, _RPW)])

    return k(x_flat)


def kernel(output):
    flat = output.reshape(_ROWS, _COLS)
    idx = _argmax_rows_sc(flat)
    return idx.reshape(64, 16, 1)
